# baseline (speedup 1.0000x reference)
import functools

import jax
import jax.numpy as jnp
from jax import lax
from jax.experimental import pallas as pl
from jax.experimental.pallas import tpu as pltpu
from jax.experimental.pallas import tpu_sc as plsc

NUM_CORES = 2
NUM_SUBCORES = 16
NW = NUM_CORES * NUM_SUBCORES
BATCH = 4096
RPW = BATCH // NW
PAIRS = RPW // 2
EMBED = 64
HIST = 50
WISH = 20
TAGS = 5
LANES = 16
NCH = EMBED // LANES


K = 8
NGROUPS = RPW // K


def _sc_pool_body(hist_hbm, wish_hbm, tags_hbm, bid_hbm, auth_hbm, lang_hbm,
                  book_hbm, aemb_hbm, lemb_hbm, temb_hbm,
                  u0_hbm, item_hbm,
                  hist_idx, wish_idx, tags_idx, bid_idx, auth_idx, lang_idx,
                  hb0, wb0, tb0, bb0, ab0, lb0,
                  hb1, wb1, tb1, bb1, ab1, lb1,
                  u0_v, item_v, sem0, sem1):
    wid = lax.axis_index("s") * NUM_CORES + lax.axis_index("c")
    base = wid * RPW
    sets = ((hb0, wb0, tb0, bb0, ab0, lb0, sem0),
            (hb1, wb1, tb1, bb1, ab1, lb1, sem1))

    pltpu.sync_copy(hist_hbm.at[pl.ds(base, RPW)], hist_idx)
    pltpu.sync_copy(wish_hbm.at[pl.ds(base, RPW)], wish_idx)
    pltpu.sync_copy(tags_hbm.at[pl.ds(base, RPW)], tags_idx)
    pltpu.sync_copy(bid_hbm.at[pl.ds(base, RPW)], bid_idx)
    pltpu.sync_copy(auth_hbm.at[pl.ds(base, RPW)], auth_idx)
    pltpu.sync_copy(lang_hbm.at[pl.ds(base, RPW)], lang_idx)

    def copies(g, s):
        hb, wb, tb, bb, ab, lb, sem = sets[s]
        out = []
        for k in range(K):
            r = g * K + k
            out.append((book_hbm.at[hist_idx.at[r]],
                        hb.at[pl.ds(k * HIST, HIST)], sem))
            out.append((book_hbm.at[wish_idx.at[r]],
                        wb.at[pl.ds(k * WISH, WISH)], sem))
            out.append((temb_hbm.at[tags_idx.at[r]],
                        tb.at[pl.ds(k * TAGS, TAGS)], sem))
        out.append((book_hbm.at[bid_idx.at[pl.ds(g * K, K)]], bb, sem))
        out.append((aemb_hbm.at[auth_idx.at[pl.ds(g * K, K)]], ab, sem))
        out.append((lemb_hbm.at[lang_idx.at[pl.ds(g * K, K)]], lb, sem))
        return out

    def fire(g, s):
        for src, dst, sem in copies(g, s):
            pltpu.async_copy(src, dst, sem)

    def drain(s):
        hb, wb, tb, bb, ab, lb, sem = sets[s]
        pltpu.make_async_copy(book_hbm.at[pl.ds(0, K * HIST)], hb, sem).wait()
        pltpu.make_async_copy(book_hbm.at[pl.ds(0, K * WISH)], wb, sem).wait()
        pltpu.make_async_copy(book_hbm.at[pl.ds(0, K * TAGS)], tb, sem).wait()
        pltpu.make_async_copy(book_hbm.at[pl.ds(0, K)], bb, sem).wait()
        pltpu.make_async_copy(book_hbm.at[pl.ds(0, K)], ab, sem).wait()
        pltpu.make_async_copy(book_hbm.at[pl.ds(0, K)], lb, sem).wait()

    def compute(g, s):
        hb, wb, tb, bb, ab, lb, _ = sets[s]

        @pl.loop(0, K)
        def _k(k):
            r = g * K + k
            sls = [pl.ds(c * LANES, LANES) for c in range(NCH)]
            acc0 = [hb[k * HIST, sl] for sl in sls]
            acc1 = [hb[k * HIST + 1, sl] for sl in sls]
            for j in range(2, HIST, 2):
                for c in range(NCH):
                    acc0[c] = acc0[c] + hb[k * HIST + j, sls[c]]
            for j in range(3, HIST, 2):
                for c in range(NCH):
                    acc1[c] = acc1[c] + hb[k * HIST + j, sls[c]]
            acc_h = [acc0[c] + acc1[c] for c in range(NCH)]
            acc0 = [wb[k * WISH, sl] for sl in sls]
            acc1 = [wb[k * WISH + 1, sl] for sl in sls]
            for j in range(2, WISH, 2):
                for c in range(NCH):
                    acc0[c] = acc0[c] + wb[k * WISH + j, sls[c]]
            for j in range(3, WISH, 2):
                for c in range(NCH):
                    acc1[c] = acc1[c] + wb[k * WISH + j, sls[c]]
            acc_w = [acc0[c] + acc1[c] for c in range(NCH)]
            acc_t = [tb[k * TAGS, sl] for sl in sls]
            for j in range(1, TAGS):
                for c in range(NCH):
                    acc_t[c] = acc_t[c] + tb[k * TAGS + j, sls[c]]
            for c in range(NCH):
                u0_v[r, sls[c]] = (acc_h[c] * (1.0 / HIST)
                                   + acc_w[c] * (1.0 / WISH))
                item_v[r, sls[c]] = (bb[k, sls[c]] + ab[k, sls[c]]
                                     + lb[k, sls[c]]
                                     + acc_t[c] * (1.0 / TAGS))

    fire(0, 0)

    @pl.loop(0, NGROUPS, step=2)
    def _grp(g):
        fire(g + 1, 1)
        drain(0)
        compute(g, 0)

        @pl.when(g + 2 < NGROUPS)
        def _():
            fire(g + 2, 0)

        drain(1)
        compute(g + 1, 1)

    pltpu.sync_copy(u0_v, u0_hbm.at[pl.ds(base, RPW)])
    pltpu.sync_copy(item_v, item_hbm.at[pl.ds(base, RPW)])


_sc_pool = functools.partial(
    pl.kernel,
    out_type=(jax.ShapeDtypeStruct((BATCH, EMBED), jnp.float32),
              jax.ShapeDtypeStruct((BATCH, EMBED), jnp.float32)),
    mesh=plsc.VectorSubcoreMesh(core_axis_name="c", subcore_axis_name="s"),
    compiler_params=pltpu.CompilerParams(use_tc_tiling_on_sc=False),
    scratch_types=[
        pltpu.VMEM((RPW, HIST), jnp.int32),
        pltpu.VMEM((RPW, WISH), jnp.int32),
        pltpu.VMEM((RPW, TAGS), jnp.int32),
        pltpu.VMEM((RPW,), jnp.int32),
        pltpu.VMEM((RPW,), jnp.int32),
        pltpu.VMEM((RPW,), jnp.int32),
    ] + 2 * [
        pltpu.VMEM((K * HIST, EMBED), jnp.float32),
        pltpu.VMEM((K * WISH, EMBED), jnp.float32),
        pltpu.VMEM((K * TAGS, EMBED), jnp.float32),
        pltpu.VMEM((K, EMBED), jnp.float32),
        pltpu.VMEM((K, EMBED), jnp.float32),
        pltpu.VMEM((K, EMBED), jnp.float32),
    ] + [
        pltpu.VMEM((RPW, EMBED), jnp.float32),
        pltpu.VMEM((RPW, EMBED), jnp.float32),
        pltpu.SemaphoreType.DMA,
        pltpu.SemaphoreType.DMA,
    ],
)(_sc_pool_body)


def _tc_towers_body(u0_ref, item_ref, dense_ref,
                    uW1, ub1, uW2, ub2, uW3, ub3,
                    dW1, db1, dW2, db2, out_ref):
    u0 = u0_ref[...]
    h = jnp.maximum(jnp.dot(u0, uW1[...],
                            preferred_element_type=jnp.float32) + ub1[...], 0.0)
    h = jnp.maximum(jnp.dot(h, uW2[...],
                            preferred_element_type=jnp.float32) + ub2[...], 0.0)
    u_emb = jnp.dot(h, uW3[...], preferred_element_type=jnp.float32) + ub3[...]
    d = jnp.maximum(jnp.dot(dense_ref[...], dW1[...],
                            preferred_element_type=jnp.float32) + db1[...], 0.0)
    d_e = jnp.dot(d, dW2[...], preferred_element_type=jnp.float32) + db2[...]
    i_emb = item_ref[...] + d_e
    out_ref[...] = jnp.sum(u_emb * i_emb, axis=1, keepdims=True)


def kernel(hist_ids, wish_ids, bid, auth, lang, tags, dense,
           book_emb, auth_emb, lang_emb, tag_emb,
           dense_W1, dense_b1, dense_W2, dense_b2,
           user_W1, user_b1, user_W2, user_b2, user_W3, user_b3):
    u0, item_part = _sc_pool(
        hist_ids.astype(jnp.int32), wish_ids.astype(jnp.int32),
        tags.astype(jnp.int32),
        bid.astype(jnp.int32), auth.astype(jnp.int32), lang.astype(jnp.int32),
        book_emb, auth_emb, lang_emb, tag_emb)

    dense_pad = jnp.concatenate(
        [dense, jnp.zeros((BATCH, 5), jnp.float32)], axis=1)
    dW1_pad = jnp.concatenate(
        [dense_W1, jnp.zeros((5, EMBED), jnp.float32)], axis=0)

    out = pl.pallas_call(
        _tc_towers_body,
        out_shape=jax.ShapeDtypeStruct((BATCH, 1), jnp.float32),
    )(u0, item_part, dense_pad,
      user_W1, user_b1.reshape(1, -1), user_W2, user_b2.reshape(1, -1),
      user_W3, user_b3.reshape(1, -1),
      dW1_pad, dense_b1.reshape(1, -1), dense_W2, dense_b2.reshape(1, -1))
    return out


# speedup vs baseline: 1.0353x; 1.0353x over previous
import functools

import jax
import jax.numpy as jnp
from jax import lax
from jax.experimental import pallas as pl
from jax.experimental.pallas import tpu as pltpu
from jax.experimental.pallas import tpu_sc as plsc

NUM_CORES = 2
NUM_SUBCORES = 16
NW = NUM_CORES * NUM_SUBCORES
BATCH = 4096
RPW = BATCH // NW
PAIRS = RPW // 2
EMBED = 64
HIST = 50
WISH = 20
TAGS = 5
LANES = 16
NCH = EMBED // LANES


K = 8
NGROUPS = RPW // K


def _sc_pool_body(hist_hbm, wish_hbm, tags_hbm, bid_hbm, auth_hbm, lang_hbm,
                  book_hbm, aemb_hbm, lemb_hbm, temb_hbm,
                  u0_hbm, item_hbm,
                  hist_idx, wish_idx, tags_idx, bid_idx, auth_idx, lang_idx,
                  hb0, wb0, tb0, bb0, ab0, lb0,
                  hb1, wb1, tb1, bb1, ab1, lb1,
                  u0_v, item_v, sem0, sem1):
    wid = lax.axis_index("s") * NUM_CORES + lax.axis_index("c")
    base = wid * RPW
    sets = ((hb0, wb0, tb0, bb0, ab0, lb0, sem0),
            (hb1, wb1, tb1, bb1, ab1, lb1, sem1))

    pltpu.sync_copy(hist_hbm.at[pl.ds(base, RPW)], hist_idx)
    pltpu.sync_copy(wish_hbm.at[pl.ds(base, RPW)], wish_idx)
    pltpu.sync_copy(tags_hbm.at[pl.ds(base, RPW)], tags_idx)
    pltpu.sync_copy(bid_hbm.at[pl.ds(base, RPW)], bid_idx)
    pltpu.sync_copy(auth_hbm.at[pl.ds(base, RPW)], auth_idx)
    pltpu.sync_copy(lang_hbm.at[pl.ds(base, RPW)], lang_idx)

    def copies(g, s):
        hb, wb, tb, bb, ab, lb, sem = sets[s]
        out = []
        for k in range(K):
            r = g * K + k
            out.append((book_hbm.at[hist_idx.at[r]],
                        hb.at[pl.ds(k * HIST, HIST)], sem))
            out.append((book_hbm.at[wish_idx.at[r]],
                        wb.at[pl.ds(k * WISH, WISH)], sem))
            out.append((temb_hbm.at[tags_idx.at[r]],
                        tb.at[pl.ds(k * TAGS, TAGS)], sem))
        out.append((book_hbm.at[bid_idx.at[pl.ds(g * K, K)]], bb, sem))
        out.append((aemb_hbm.at[auth_idx.at[pl.ds(g * K, K)]], ab, sem))
        out.append((lemb_hbm.at[lang_idx.at[pl.ds(g * K, K)]], lb, sem))
        return out

    def fire(g, s):
        for src, dst, sem in copies(g, s):
            pltpu.async_copy(src, dst, sem)

    def drain(g, s):
        for src, dst, sem in copies(g, s):
            pltpu.make_async_copy(src, dst, sem).wait()

    def compute(g, s):
        hb, wb, tb, bb, ab, lb, _ = sets[s]

        @pl.loop(0, K)
        def _k(k):
            r = g * K + k
            sls = [pl.ds(c * LANES, LANES) for c in range(NCH)]
            acc_h = [hb[k * HIST, sl] for sl in sls]
            for j in range(1, HIST):
                for c in range(NCH):
                    acc_h[c] = acc_h[c] + hb[k * HIST + j, sls[c]]
            acc_w = [wb[k * WISH, sl] for sl in sls]
            for j in range(1, WISH):
                for c in range(NCH):
                    acc_w[c] = acc_w[c] + wb[k * WISH + j, sls[c]]
            acc_t = [tb[k * TAGS, sl] for sl in sls]
            for j in range(1, TAGS):
                for c in range(NCH):
                    acc_t[c] = acc_t[c] + tb[k * TAGS + j, sls[c]]
            for c in range(NCH):
                u0_v[r, sls[c]] = (acc_h[c] * (1.0 / HIST)
                                   + acc_w[c] * (1.0 / WISH))
                item_v[r, sls[c]] = (bb[k, sls[c]] + ab[k, sls[c]]
                                     + lb[k, sls[c]]
                                     + acc_t[c] * (1.0 / TAGS))

    fire(0, 0)

    @pl.loop(0, NGROUPS, step=2)
    def _grp(g):
        fire(g + 1, 1)
        drain(g, 0)
        compute(g, 0)

        @pl.when(g + 2 < NGROUPS)
        def _():
            fire(g + 2, 0)

        drain(g + 1, 1)
        compute(g + 1, 1)

    pltpu.sync_copy(u0_v, u0_hbm.at[pl.ds(base, RPW)])
    pltpu.sync_copy(item_v, item_hbm.at[pl.ds(base, RPW)])


_sc_pool = functools.partial(
    pl.kernel,
    out_type=(jax.ShapeDtypeStruct((BATCH, EMBED), jnp.float32),
              jax.ShapeDtypeStruct((BATCH, EMBED), jnp.float32)),
    mesh=plsc.VectorSubcoreMesh(core_axis_name="c", subcore_axis_name="s"),
    compiler_params=pltpu.CompilerParams(use_tc_tiling_on_sc=False),
    scratch_types=[
        pltpu.VMEM((RPW, HIST), jnp.int32),
        pltpu.VMEM((RPW, WISH), jnp.int32),
        pltpu.VMEM((RPW, TAGS), jnp.int32),
        pltpu.VMEM((RPW,), jnp.int32),
        pltpu.VMEM((RPW,), jnp.int32),
        pltpu.VMEM((RPW,), jnp.int32),
    ] + 2 * [
        pltpu.VMEM((K * HIST, EMBED), jnp.float32),
        pltpu.VMEM((K * WISH, EMBED), jnp.float32),
        pltpu.VMEM((K * TAGS, EMBED), jnp.float32),
        pltpu.VMEM((K, EMBED), jnp.float32),
        pltpu.VMEM((K, EMBED), jnp.float32),
        pltpu.VMEM((K, EMBED), jnp.float32),
    ] + [
        pltpu.VMEM((RPW, EMBED), jnp.float32),
        pltpu.VMEM((RPW, EMBED), jnp.float32),
        pltpu.SemaphoreType.DMA,
        pltpu.SemaphoreType.DMA,
    ],
)(_sc_pool_body)


def _tc_towers_body(u0_ref, item_ref, dense_ref,
                    uW1, ub1, uW2, ub2, uW3, ub3,
                    dW1, db1, dW2, db2, out_ref):
    u0 = u0_ref[...]
    h = jnp.maximum(jnp.dot(u0, uW1[...],
                            preferred_element_type=jnp.float32) + ub1[...], 0.0)
    h = jnp.maximum(jnp.dot(h, uW2[...],
                            preferred_element_type=jnp.float32) + ub2[...], 0.0)
    u_emb = jnp.dot(h, uW3[...], preferred_element_type=jnp.float32) + ub3[...]
    d = jnp.maximum(jnp.dot(dense_ref[...], dW1[...],
                            preferred_element_type=jnp.float32) + db1[...], 0.0)
    d_e = jnp.dot(d, dW2[...], preferred_element_type=jnp.float32) + db2[...]
    i_emb = item_ref[...] + d_e
    out_ref[...] = jnp.sum(u_emb * i_emb, axis=1, keepdims=True)


def kernel(hist_ids, wish_ids, bid, auth, lang, tags, dense,
           book_emb, auth_emb, lang_emb, tag_emb,
           dense_W1, dense_b1, dense_W2, dense_b2,
           user_W1, user_b1, user_W2, user_b2, user_W3, user_b3):
    u0, item_part = _sc_pool(
        hist_ids.astype(jnp.int32), wish_ids.astype(jnp.int32),
        tags.astype(jnp.int32),
        bid.astype(jnp.int32), auth.astype(jnp.int32), lang.astype(jnp.int32),
        book_emb, auth_emb, lang_emb, tag_emb)

    dense_pad = jnp.concatenate(
        [dense, jnp.zeros((BATCH, 5), jnp.float32)], axis=1)
    dW1_pad = jnp.concatenate(
        [dense_W1, jnp.zeros((5, EMBED), jnp.float32)], axis=0)

    out = pl.pallas_call(
        _tc_towers_body,
        out_shape=jax.ShapeDtypeStruct((BATCH, 1), jnp.float32),
    )(u0, item_part, dense_pad,
      user_W1, user_b1.reshape(1, -1), user_W2, user_b2.reshape(1, -1),
      user_W3, user_b3.reshape(1, -1),
      dW1_pad, dense_b1.reshape(1, -1), dense_W2, dense_b2.reshape(1, -1))
    return out
